# baseline (speedup 1.0000x reference)
import functools

import jax
import jax.numpy as jnp
from jax import lax
from jax.experimental import pallas as pl
from jax.experimental.pallas import tpu as pltpu
from jax.experimental.pallas import tpu_sc as plsc

BATCH = 1024
WORDS = 200
COORDS = 64
NUM_WORKERS = 32
BQ = 256
NQ = BATCH // BQ
N_BLOCKS = WORDS * NQ
BLOCKS_PER_W = N_BLOCKS // NUM_WORKERS
IDX_MINOR = 128


def kernel(sequence_bw, vocab_table_tc, pos_table_wc):
    seq_p = sequence_bw.T
    pos_p = pos_table_wc.T
    mesh = plsc.VectorSubcoreMesh(core_axis_name="c", subcore_axis_name="s")

    @functools.partial(
        pl.kernel,
        out_type=jax.ShapeDtypeStruct((WORDS, COORDS, BATCH), jnp.float32),
        mesh=mesh,
        scratch_types=[
            pltpu.VMEM((BQ,), jnp.int32),
            pltpu.VMEM((BQ, COORDS), jnp.float32),
            pltpu.VMEM((COORDS, BQ), jnp.float32),
            pltpu.VMEM((COORDS, WORDS), jnp.float32),
            pltpu.SemaphoreType.DMA,
        ],
        compiler_params=pltpu.CompilerParams(
            use_tc_tiling_on_sc=False, needs_layout_passes=False
        ),
    )
    def sc_kernel(seq_hbm, table_hbm, pos_hbm, out_hbm,
                  idx_v, gath_v, stage_v, pos_v, sem):
        wid = lax.axis_index("s") * 2 + lax.axis_index("c")
        pltpu.sync_copy(pos_hbm, pos_v)
        rows16 = [lax.iota(jnp.int32, 16) + (b16 * 16) for b16 in range(16)]

        @pl.loop(0, BLOCKS_PER_W)
        def _block(j):
            blk = wid * BLOCKS_PER_W + j
            w = blk // NQ
            q = blk % NQ
            pltpu.sync_copy(seq_hbm.at[w, pl.ds(q * BQ, BQ)], idx_v)
            for g in range(BQ // IDX_MINOR):
                pltpu.async_copy(
                    table_hbm.at[idx_v.at[pl.ds(g * IDX_MINOR, IDX_MINOR)]],
                    gath_v.at[pl.ds(g * IDX_MINOR, IDX_MINOR)],
                    sem,
                ).wait()

            w_vec = jnp.full((16,), 0, jnp.int32) + w

            @pl.loop(0, COORDS)
            def _coord(c):
                col_c = jnp.full((16,), 0, jnp.int32) + c
                pos_vec = plsc.load_gather(pos_v, [col_c, w_vec])
                for b16 in range(16):
                    vec = plsc.load_gather(gath_v, [rows16[b16], col_c])
                    stage_v[c, pl.ds(b16 * 16, 16)] = vec + pos_vec

            pltpu.sync_copy(stage_v, out_hbm.at[w, :, pl.ds(q * BQ, BQ)])

    out_p = sc_kernel(seq_p, vocab_table_tc, pos_p)
    return out_p.transpose(2, 0, 1)


# speedup vs baseline: 1.2731x; 1.2731x over previous
import functools

import jax
import jax.numpy as jnp
from jax import lax
from jax.experimental import pallas as pl
from jax.experimental.pallas import tpu as pltpu
from jax.experimental.pallas import tpu_sc as plsc

BATCH = 1024
WORDS = 200
COORDS = 64
NUM_WORKERS = 32
ROWS_TOTAL = BATCH * WORDS
ROWS_PER_W = ROWS_TOTAL // NUM_WORKERS
CHUNK = 256
IDX_MINOR = 128
N_CHUNKS = ROWS_PER_W // CHUNK
NBUF = 3


def kernel(sequence_bw, vocab_table_tc, pos_table_wc):
    seq_flat = sequence_bw.reshape(ROWS_TOTAL)
    mesh = plsc.VectorSubcoreMesh(core_axis_name="c", subcore_axis_name="s")

    @functools.partial(
        pl.kernel,
        out_type=jax.ShapeDtypeStruct((ROWS_TOTAL, COORDS), jnp.float32),
        mesh=mesh,
        scratch_types=[
            pltpu.VMEM((NBUF, CHUNK), jnp.int32),
            pltpu.VMEM((NBUF, CHUNK, COORDS), jnp.float32),
            pltpu.VMEM((WORDS, COORDS), jnp.float32),
            [pltpu.SemaphoreType.DMA] * NBUF,
            [pltpu.SemaphoreType.DMA] * NBUF,
        ],
        compiler_params=pltpu.CompilerParams(
            use_tc_tiling_on_sc=False, needs_layout_passes=False
        ),
    )
    def sc_kernel(seq_hbm, table_hbm, pos_hbm, out_hbm,
                  idx_v, rows_v, pos_v, gsems, ssems):
        wid = lax.axis_index("s") * 2 + lax.axis_index("c")
        pltpu.sync_copy(pos_hbm, pos_v)
        base0 = wid * ROWS_PER_W

        def fire(j, buf):
            base = base0 + j * CHUNK
            pltpu.sync_copy(seq_hbm.at[pl.ds(base, CHUNK)], idx_v.at[buf])
            for g in range(CHUNK // IDX_MINOR):
                pltpu.async_copy(
                    table_hbm.at[idx_v.at[buf, pl.ds(g * IDX_MINOR, IDX_MINOR)]],
                    rows_v.at[buf, pl.ds(g * IDX_MINOR, IDX_MINOR)],
                    gsems[buf],
                )

        def finish(j, buf):
            base = base0 + j * CHUNK
            for g in range(CHUNK // IDX_MINOR):
                pltpu.make_async_copy(
                    table_hbm.at[idx_v.at[buf, pl.ds(g * IDX_MINOR, IDX_MINOR)]],
                    rows_v.at[buf, pl.ds(g * IDX_MINOR, IDX_MINOR)],
                    gsems[buf],
                ).wait()

            @pl.loop(0, CHUNK)
            def _row(i):
                w = lax.rem(base + i, WORDS)
                for c in range(COORDS // 16):
                    plsc.addupdate(
                        rows_v.at[buf, i, pl.ds(c * 16, 16)],
                        pos_v[w, pl.ds(c * 16, 16)],
                    )

            pltpu.async_copy(
                rows_v.at[buf], out_hbm.at[pl.ds(base, CHUNK)], ssems[buf]
            )

        def wait_store(j, buf):
            base = base0 + j * CHUNK
            pltpu.make_async_copy(
                rows_v.at[buf], out_hbm.at[pl.ds(base, CHUNK)], ssems[buf]
            ).wait()

        fire(0, 0)
        fire(1, 1)
        finish(0, 0)
        fire(2, 2)
        finish(1, 1)
        wait_store(0, 0)
        fire(3, 0)
        finish(2, 2)
        wait_store(1, 1)
        fire(4, 1)

        @pl.loop(0, 6)
        def _steady(k):
            j = 3 + 3 * k
            finish(j, 0)
            wait_store(j - 1, 2)
            fire(j + 2, 2)
            finish(j + 1, 1)
            wait_store(j, 0)
            fire(j + 3, 0)
            finish(j + 2, 2)
            wait_store(j + 1, 1)
            fire(j + 4, 1)

        finish(21, 0)
        wait_store(20, 2)
        fire(23, 2)
        finish(22, 1)
        wait_store(21, 0)
        fire(24, 0)
        finish(23, 2)
        wait_store(22, 1)
        finish(24, 0)
        wait_store(23, 2)
        wait_store(24, 0)

    out = sc_kernel(seq_flat, vocab_table_tc, pos_table_wc)
    return out.reshape(BATCH, WORDS, COORDS)


# speedup vs baseline: 1.3625x; 1.0703x over previous
import functools

import jax
import jax.numpy as jnp
from jax import lax
from jax.experimental import pallas as pl
from jax.experimental.pallas import tpu as pltpu
from jax.experimental.pallas import tpu_sc as plsc

BATCH = 1024
WORDS = 200
COORDS = 64
NUM_WORKERS = 32
ROWS_PER_W = BATCH // NUM_WORKERS
IDX_MINOR = 128
NBUF = 3
GATHER_SPLITS = ((0, IDX_MINOR), (IDX_MINOR, WORDS - IDX_MINOR))


def kernel(sequence_bw, vocab_table_tc, pos_table_wc):
    mesh = plsc.VectorSubcoreMesh(core_axis_name="c", subcore_axis_name="s")

    @functools.partial(
        pl.kernel,
        out_type=jax.ShapeDtypeStruct((BATCH, WORDS, COORDS), jnp.float32),
        mesh=mesh,
        scratch_types=[
            pltpu.VMEM((NBUF, WORDS), jnp.int32),
            pltpu.VMEM((NBUF, WORDS, COORDS), jnp.float32),
            pltpu.VMEM((WORDS, COORDS), jnp.float32),
            [pltpu.SemaphoreType.DMA] * NBUF,
            [pltpu.SemaphoreType.DMA] * NBUF,
        ],
        compiler_params=pltpu.CompilerParams(
            use_tc_tiling_on_sc=False, needs_layout_passes=False
        ),
    )
    def sc_kernel(seq_hbm, table_hbm, pos_hbm, out_hbm,
                  idx_v, rows_v, pos_v, gsems, ssems):
        wid = lax.axis_index("s") * 2 + lax.axis_index("c")
        pltpu.sync_copy(pos_hbm, pos_v)
        b0 = wid * ROWS_PER_W

        def fire(j, buf):
            pltpu.sync_copy(seq_hbm.at[b0 + j], idx_v.at[buf])
            for off, n in GATHER_SPLITS:
                pltpu.async_copy(
                    table_hbm.at[idx_v.at[buf, pl.ds(off, n)]],
                    rows_v.at[buf, pl.ds(off, n)],
                    gsems[buf],
                )

        def finish(j, buf):
            for off, n in GATHER_SPLITS:
                pltpu.make_async_copy(
                    table_hbm.at[idx_v.at[buf, pl.ds(off, n)]],
                    rows_v.at[buf, pl.ds(off, n)],
                    gsems[buf],
                ).wait()

            @pl.loop(0, WORDS, step=2)
            def _row(i):
                for u in range(2):
                    for c in range(COORDS // 16):
                        plsc.addupdate(
                            rows_v.at[buf, i + u, pl.ds(c * 16, 16)],
                            pos_v[i + u, pl.ds(c * 16, 16)],
                        )

            pltpu.async_copy(rows_v.at[buf], out_hbm.at[b0 + j], ssems[buf])

        def wait_store(j, buf):
            pltpu.make_async_copy(
                rows_v.at[buf], out_hbm.at[b0 + j], ssems[buf]
            ).wait()

        fire(0, 0)
        fire(1, 1)
        for j in range(ROWS_PER_W):
            finish(j, j % NBUF)
            if j >= 1:
                wait_store(j - 1, (j - 1) % NBUF)
            if j + 2 < ROWS_PER_W:
                fire(j + 2, (j + 2) % NBUF)
        wait_store(ROWS_PER_W - 1, (ROWS_PER_W - 1) % NBUF)

    return sc_kernel(sequence_bw, vocab_table_tc, pos_table_wc)


# speedup vs baseline: 1.4336x; 1.0521x over previous
import functools

import jax
import jax.numpy as jnp
from jax import lax
from jax.experimental import pallas as pl
from jax.experimental.pallas import tpu as pltpu
from jax.experimental.pallas import tpu_sc as plsc

BATCH = 1024
WORDS = 200
COORDS = 64
TOKENS = 1000000
NUM_WORKERS = 32
W_PARTS = 4
B_COLS = 8
W_PER_TILE = WORDS // W_PARTS
BW = 128
STAGE_ROWS = 56
GPITCH = 128


def kernel(sequence_bw, vocab_table_tc, pos_table_wc):
    seq_p = sequence_bw.T
    pos_p = pos_table_wc.T
    table2 = vocab_table_tc.reshape(TOKENS // 2, 2 * COORDS)
    mesh = plsc.VectorSubcoreMesh(core_axis_name="c", subcore_axis_name="s")

    @functools.partial(
        pl.kernel,
        out_type=jax.ShapeDtypeStruct((WORDS, COORDS, BATCH), jnp.float32),
        mesh=mesh,
        scratch_types=[
            pltpu.VMEM((STAGE_ROWS, BW), jnp.int32),
            pltpu.VMEM((2, BW), jnp.int32),
            pltpu.VMEM((2, BW, GPITCH), jnp.float32),
            pltpu.VMEM((2, COORDS, BW), jnp.float32),
            pltpu.VMEM((COORDS, WORDS), jnp.float32),
            [pltpu.SemaphoreType.DMA] * 2,
            [pltpu.SemaphoreType.DMA] * 2,
        ],
        compiler_params=pltpu.CompilerParams(
            use_tc_tiling_on_sc=True, needs_layout_passes=False
        ),
    )
    def sc_kernel(seq_hbm, table_hbm, pos_hbm, out_hbm,
                  idxs_v, gidx_v, gath_v, ostage_v, pos_v, gsems, ssems):
        wid = lax.axis_index("s") * 2 + lax.axis_index("c")
        w_part = wid // B_COLS
        b0 = pl.multiple_of((wid % B_COLS) * BW, BW)
        w0 = w_part * W_PER_TILE
        w_lo = pl.multiple_of(
            w_part * W_PER_TILE - (w_part * W_PER_TILE) % 8, 8
        )

        pltpu.sync_copy(pos_hbm, pos_v)
        pltpu.sync_copy(
            seq_hbm.at[pl.ds(w_lo, STAGE_ROWS), pl.ds(b0, BW)], idxs_v
        )
        iota = lax.iota(jnp.int32, 16)

        def prep_and_fire(u, buf):
            r = w0 - w_lo + u
            for k in range(BW // 16):
                v = idxs_v[r, pl.ds(k * 16, 16)]
                gidx_v[buf, pl.ds(k * 16, 16)] = v >> 1
            pltpu.async_copy(
                table_hbm.at[gidx_v.at[buf]],
                gath_v.at[buf, :, pl.ds(0, 2 * COORDS)],
                gsems[buf],
            )

        def wait_gather(buf):
            pltpu.make_async_copy(
                table_hbm.at[gidx_v.at[buf]],
                gath_v.at[buf, :, pl.ds(0, 2 * COORDS)],
                gsems[buf],
            ).wait()

        def compute(u, buf):
            r = w0 - w_lo + u
            w_vec = jnp.full((16,), 0, jnp.int32) + (w0 + u)
            h16 = []
            for j in range(BW // 16):
                v = idxs_v[r, pl.ds(j * 16, 16)]
                h16.append((v & 1) << 6)
            pv = [
                plsc.load_gather(pos_v, [k * 16 + iota, w_vec])
                for k in range(COORDS // 16)
            ]
            gref = gath_v.at[buf]
            oref = ostage_v.at[buf]
            @pl.loop(0, 16)
            def _diag(d):
                rot = (iota + d) & 15
                for k in range(COORDS // 16):
                    rotc = rot + (k * 16)
                    posr = pv[k].at[rot].get(mode="promise_in_bounds")
                    for j in range(BW // 16):
                        rows = j * 16 + iota
                        vec = plsc.load_gather(gref, [rows, h16[j] + rotc])
                        plsc.store_scatter(oref, [rotc, rows], vec + posr)

        def fire_store(u, buf):
            pltpu.async_copy(
                ostage_v.at[buf],
                out_hbm.at[w0 + u, :, pl.ds(b0, BW)],
                ssems[buf],
            )

        def wait_store(u, buf):
            pltpu.make_async_copy(
                ostage_v.at[buf],
                out_hbm.at[w0 + u, :, pl.ds(b0, BW)],
                ssems[buf],
            ).wait()

        prep_and_fire(0, 0)
        prep_and_fire(1, 1)

        @pl.loop(0, W_PER_TILE // 2)
        def _pair(k):
            u = 2 * k
            wait_gather(0)

            @pl.when(k > 0)
            def _ws0():
                wait_store(u - 2, 0)

            compute(u, 0)
            fire_store(u, 0)

            @pl.when(k < W_PER_TILE // 2 - 1)
            def _fg0():
                prep_and_fire(u + 2, 0)

            wait_gather(1)

            @pl.when(k > 0)
            def _ws1():
                wait_store(u - 1, 1)

            compute(u + 1, 1)
            fire_store(u + 1, 1)

            @pl.when(k < W_PER_TILE // 2 - 1)
            def _fg1():
                prep_and_fire(u + 3, 1)

        wait_store(W_PER_TILE - 2, 0)
        wait_store(W_PER_TILE - 1, 1)

    out_p = sc_kernel(seq_p, table2, pos_p)
    return out_p.transpose(2, 0, 1)
